# Initial kernel scaffold; baseline (speedup 1.0000x reference)
import jax
import jax.numpy as jnp
from jax.experimental import pallas as pl


def kernel(x_location, x_mobility_batch, x_text_batch, sorted_user, sorted_location):
    raise NotImplementedError("write your pallas kernel here")


# speedup vs baseline: 3.9120x; 3.9120x over previous
import jax
import jax.numpy as jnp
from jax.experimental import pallas as pl
from jax.experimental.pallas import tpu as pltpu


def _round_up(x, m):
    return ((x + m - 1) // m) * m

_GATHER_W = 8


def _gather_kernel(lidx_ref, *refs):
    del lidx_ref
    xrows, out_ref = refs[:_GATHER_W], refs[_GATHER_W]
    for j in range(_GATHER_W):
        out_ref[j, :] = xrows[j][0, 0, :]


def _batch_kernel(uidx_ref, lemb_ref, out_ref, *, n_user, d_col):
    tl = lemb_ref.shape[0]
    rows = jax.lax.broadcasted_iota(jnp.int32, (n_user, tl), 0)
    oh = (rows == uidx_ref[0]).astype(jnp.float32)
    sums = jnp.dot(oh, lemb_ref[...], preferred_element_type=jnp.float32)
    counts = sums[:, d_col:d_col + 1]
    has = counts > 0.0
    avg = sums / jnp.maximum(counts, 1.0)
    n_edge = jnp.maximum(jnp.sum(has.astype(jnp.float32)), 1.0)
    mean_b = jnp.sum(avg, axis=0, keepdims=True) / n_edge
    out_ref[...] = jnp.where(has, avg, mean_b)


def kernel(x_location, x_mobility_batch, x_text_batch, sorted_user, sorted_location):
    x_m_t = jnp.concatenate([x_mobility_batch, x_text_batch], axis=2)
    links0 = x_m_t[:, 0]
    batch, L, _ = links0.shape
    n_loc, D = x_location.shape
    n_user = sorted_user.shape[0]

    uidx = jnp.take(sorted_user, links0[..., 0]).astype(jnp.int32)
    lidx = jnp.take(sorted_location, links0[..., 1]).astype(jnp.int32)
    lidx_flat = lidx.reshape(batch * L)

    D_pad = 128 * pl.cdiv(D + 1, 128)
    xloc_aug = jnp.concatenate(
        [x_location.astype(jnp.float32),
         jnp.ones((n_loc, 1), jnp.float32),
         jnp.zeros((n_loc, D_pad - D - 1), jnp.float32)],
        axis=1).reshape(n_loc, 1, D_pad)

    LB = batch * L
    n_steps = LB // _GATHER_W
    in_specs = [
        pl.BlockSpec((1, 1, D_pad),
                     (lambda s, lidx_ref, j=j: (lidx_ref[_GATHER_W * s + j], 0, 0)))
        for j in range(_GATHER_W)
    ]
    link_emb = pl.pallas_call(
        _gather_kernel,
        out_shape=jax.ShapeDtypeStruct((LB, D_pad), jnp.float32),
        grid_spec=pltpu.PrefetchScalarGridSpec(
            num_scalar_prefetch=1,
            grid=(n_steps,),
            in_specs=in_specs,
            out_specs=pl.BlockSpec((_GATHER_W, D_pad), lambda s, lidx_ref: (s, 0)),
        ),
        compiler_params=pltpu.CompilerParams(
            dimension_semantics=("arbitrary",),
            vmem_limit_bytes=32 * 1024 * 1024),
    )(lidx_flat, *([xloc_aug] * _GATHER_W))

    import functools
    body = functools.partial(_batch_kernel, n_user=n_user, d_col=D)
    out2 = pl.pallas_call(
        body,
        out_shape=jax.ShapeDtypeStruct((batch * n_user, D_pad), jnp.float32),
        grid_spec=pltpu.PrefetchScalarGridSpec(
            num_scalar_prefetch=0,
            grid=(batch,),
            in_specs=[pl.BlockSpec((1, 1, L), lambda b: (b, 0, 0)),
                      pl.BlockSpec((L, D_pad), lambda b: (b, 0))],
            out_specs=pl.BlockSpec((n_user, D_pad), lambda b: (b, 0)),
        ),
        compiler_params=pltpu.CompilerParams(
            dimension_semantics=("parallel",),
            vmem_limit_bytes=64 * 1024 * 1024),
    )(uidx.reshape(batch, 1, L), link_emb)

    out3 = out2.reshape(batch, n_user, D_pad)[:, :, :D]
    return [out3[i] for i in range(batch)]
